# Initial kernel scaffold; baseline (speedup 1.0000x reference)
import jax
import jax.numpy as jnp
from jax.experimental import pallas as pl


def kernel(det_rois, det_labels, det_scores, im_info):
    raise NotImplementedError("write your pallas kernel here")


# speedup vs baseline: 13.8303x; 13.8303x over previous
import functools

import jax
import jax.numpy as jnp
from jax import lax
from jax.experimental import pallas as pl
from jax.experimental.pallas import tpu as pltpu
from jax.experimental.pallas import tpu_sc as plsc

N = 512
R = N - 1
P = N * R
W5 = 5 * R
NC, NS = 2, 16
NW = NC * NS
ROWS_PER_W = N // NW
GROUP = 8
RGRP = GROUP * W5
SGRP = GROUP * R


def _sc_pairs(rois_flat, labels, scores):
    mesh = plsc.VectorSubcoreMesh(core_axis_name="c", subcore_axis_name="s",
                                  num_cores=NC, num_subcores=NS)
    f32, i32 = jnp.float32, jnp.int32

    @functools.partial(
        pl.kernel,
        out_type=(
            jax.ShapeDtypeStruct((N * W5,), f32),
            jax.ShapeDtypeStruct((N * W5,), f32),
            jax.ShapeDtypeStruct((N * W5,), f32),
            jax.ShapeDtypeStruct((P,), i32),
            jax.ShapeDtypeStruct((P,), i32),
            jax.ShapeDtypeStruct((P,), f32),
            jax.ShapeDtypeStruct((P,), f32),
            jax.ShapeDtypeStruct((P,), i32),
            jax.ShapeDtypeStruct((P,), i32),
        ),
        mesh=mesh,
        compiler_params=pltpu.CompilerParams(needs_layout_passes=False),
        scratch_types=(
            pltpu.VMEM((2576,), f32),
            pltpu.VMEM((528,), i32),
            pltpu.VMEM((528,), f32),
            pltpu.VMEM((20448,), f32),
            pltpu.VMEM((20448,), f32),
            pltpu.VMEM((20448,), f32),
            pltpu.VMEM((4096,), i32),
            pltpu.VMEM((4096,), i32),
            pltpu.VMEM((4096,), f32),
            pltpu.VMEM((4096,), f32),
            pltpu.VMEM((4096,), i32),
            pltpu.VMEM((4096,), i32),
        ),
    )
    def k(rois_hbm, lab_hbm, sco_hbm,
          srois_out, orois_out, rrois_out,
          slab_out, olab_out, ssc_out, osc_out, sind_out, oind_out,
          detv, labv, scov, sbuf, obuf, rbuf,
          slabb, olabb, sscb, oscb, sindb, oindb):
        wid = lax.axis_index("s") * NC + lax.axis_index("c")
        pltpu.sync_copy(rois_hbm, detv.at[pl.ds(0, 5 * N)])
        pltpu.sync_copy(lab_hbm, labv.at[pl.ds(0, N)])
        pltpu.sync_copy(sco_hbm, scov.at[pl.ds(0, N)])

        lane = lax.iota(i32, 16)
        colp = [(lane + 16 * p) % 5 for p in range(5)]
        is0 = [c == 0 for c in colp]
        ismin = [(c == 1) | (c == 2) for c in colp]

        for g in range(2):
            i0 = wid * ROWS_PER_W + g * GROUP

            def row_body(t, _):
                i = i0 + t
                i5 = 5 * i
                i5v = jnp.full((16,), i5, i32)
                iv = jnp.full((16,), i, i32)
                sbjpat = [plsc.load_gather(detv, [i5v + colp[p]])
                          for p in range(5)]
                slabv = plsc.load_gather(labv, [iv])
                sscv = plsc.load_gather(scov, [iv])
                rbase = t * W5
                sbase = t * R

                def u_body(u, _):
                    w0 = u * 80
                    for p in range(5):
                        wv = (w0 + 16 * p) + lane
                        src = wv + jnp.where(wv >= i5v, 5, 0)
                        obj = plsc.load_gather(detv, [src])
                        off = rbase + w0 + 16 * p
                        obuf[pl.ds(off, 16)] = obj
                        sbuf[pl.ds(off, 16)] = sbjpat[p]
                        rel = jnp.where(
                            is0[p], sbjpat[p],
                            jnp.where(ismin[p],
                                      jnp.minimum(obj, sbjpat[p]),
                                      jnp.maximum(obj, sbjpat[p])))
                        rbuf[pl.ds(off, 16)] = rel
                    return 0

                lax.fori_loop(0, 32, u_body, 0)

                def v_body(u, _):
                    jv = u * 16 + lane
                    jp = jv + jnp.where(jv >= iv, 1, 0)
                    off = sbase + u * 16
                    olabb[pl.ds(off, 16)] = plsc.load_gather(labv, [jp])
                    oscb[pl.ds(off, 16)] = plsc.load_gather(scov, [jp])
                    oindb[pl.ds(off, 16)] = jp
                    sindb[pl.ds(off, 16)] = iv
                    slabb[pl.ds(off, 16)] = slabv
                    sscb[pl.ds(off, 16)] = sscv
                    return 0

                lax.fori_loop(0, 32, v_body, 0)
                return 0

            lax.fori_loop(0, GROUP, row_body, 0)

            roff = pl.multiple_of((wid * 2 + g) * RGRP, 8)
            soff = pl.multiple_of((wid * 2 + g) * SGRP, 8)
            pltpu.sync_copy(sbuf.at[pl.ds(0, RGRP)], srois_out.at[pl.ds(roff, RGRP)])
            pltpu.sync_copy(obuf.at[pl.ds(0, RGRP)], orois_out.at[pl.ds(roff, RGRP)])
            pltpu.sync_copy(rbuf.at[pl.ds(0, RGRP)], rrois_out.at[pl.ds(roff, RGRP)])
            pltpu.sync_copy(slabb.at[pl.ds(0, SGRP)], slab_out.at[pl.ds(soff, SGRP)])
            pltpu.sync_copy(olabb.at[pl.ds(0, SGRP)], olab_out.at[pl.ds(soff, SGRP)])
            pltpu.sync_copy(sscb.at[pl.ds(0, SGRP)], ssc_out.at[pl.ds(soff, SGRP)])
            pltpu.sync_copy(oscb.at[pl.ds(0, SGRP)], osc_out.at[pl.ds(soff, SGRP)])
            pltpu.sync_copy(sindb.at[pl.ds(0, SGRP)], sind_out.at[pl.ds(soff, SGRP)])
            pltpu.sync_copy(oindb.at[pl.ds(0, SGRP)], oind_out.at[pl.ds(soff, SGRP)])

    return k(rois_flat, labels, scores)


def kernel(det_rois, det_labels, det_scores, im_info):
    del im_info
    rois_flat = det_rois.reshape(-1)
    (srois, orois, rrois, slab, olab, ssc, osc, sind, oind) = _sc_pairs(
        rois_flat, det_labels.astype(jnp.int32), det_scores)
    fg_size = jnp.full((1,), P, jnp.int32)
    return (det_rois, sind, oind,
            srois.reshape(P, 5), orois.reshape(P, 5), rrois.reshape(P, 5),
            slab, olab, ssc, osc, fg_size)
